# Initial kernel scaffold; baseline (speedup 1.0000x reference)
import functools

import jax
import jax.numpy as jnp
from jax import lax
from jax.experimental import pallas as pl
from jax.experimental.pallas import tpu as pltpu
from jax.experimental.pallas import tpu_sc as plsc

N = 10000
NP = 10240
D = 256
HD = 128
E = 160000
E_A = 163840
E_B = 161792
BR = 1024
GRID = NP // BR

_mesh = plsc.VectorSubcoreMesh(core_axis_name="c", subcore_axis_name="s")


@functools.partial(
    pl.kernel,
    out_type=jax.ShapeDtypeStruct((2, NP), jnp.float32),
    mesh=_mesh,
    scratch_types=[
        pltpu.VMEM((40, 128), jnp.int32),
        pltpu.VMEM((128,), jnp.float32),
        pltpu.VMEM((640,), jnp.float32),
        pltpu.VMEM_SHARED((NP,), jnp.float32),
    ],
)
def _sc_counts(row_hbm, out_hbm, idx_v, ones_v, z_v, counts_sh):
    c = lax.axis_index("c")
    s = lax.axis_index("s")
    wid = s * 2 + c

    @pl.loop(0, 640, step=16)
    def _(k):
        z_v[pl.ds(k, 16)] = jnp.zeros((16,), jnp.float32)

    @pl.loop(0, 128, step=16)
    def _(k):
        ones_v[pl.ds(k, 16)] = jnp.ones((16,), jnp.float32)

    pltpu.sync_copy(z_v, counts_sh.at[pl.ds(s * 640, 640)])
    plsc.subcore_barrier()

    pltpu.sync_copy(row_hbm.at[wid], idx_v)

    @pl.loop(0, 40)
    def _(j):
        pltpu.sync_copy(ones_v, counts_sh.at[idx_v.at[j]], add=True)

    plsc.subcore_barrier()
    pltpu.sync_copy(counts_sh.at[pl.ds(s * 640, 640)],
                    out_hbm.at[c].at[pl.ds(s * 640, 640)])


@functools.partial(
    pl.kernel,
    out_type=jax.ShapeDtypeStruct((2, NP, HD), jnp.float32),
    mesh=_mesh,
    scratch_types=[
        pltpu.VMEM((79, 128), jnp.int32),
        pltpu.VMEM((79, 128), jnp.int32),
        pltpu.VMEM((128, HD), jnp.float32),
        pltpu.VMEM((64, HD), jnp.float32),
        pltpu.VMEM_SHARED((NP, HD), jnp.float32),
    ],
)
def _sc_spmm(hp0_hbm, hp1_hbm, row_hbm, col_hbm, out_hbm,
             ridx, cidx, gbuf, zbuf, accum):
    c = lax.axis_index("c")
    s = lax.axis_index("s")

    @pl.loop(0, 64)
    def _(r):
        @pl.loop(0, HD, step=16)
        def _(k):
            zbuf[r, pl.ds(k, 16)] = jnp.zeros((16,), jnp.float32)

    @pl.loop(0, 10)
    def _(t):
        pltpu.sync_copy(zbuf, accum.at[pl.ds(s * 640 + t * 64, 64)])

    plsc.subcore_barrier()

    pltpu.sync_copy(row_hbm.at[s], ridx)
    pltpu.sync_copy(col_hbm.at[s], cidx)

    def run_half(hp_hbm):
        @pl.loop(0, 79)
        def _(j):
            pltpu.sync_copy(hp_hbm.at[cidx.at[j]], gbuf)
            pltpu.sync_copy(gbuf, accum.at[ridx.at[j]], add=True)

    @pl.when(c == 0)
    def _():
        run_half(hp0_hbm)

    @pl.when(c == 1)
    def _():
        run_half(hp1_hbm)

    plsc.subcore_barrier()
    pltpu.sync_copy(accum.at[pl.ds(s * 640, 640)],
                    out_hbm.at[c].at[pl.ds(s * 640, 640)])


def _tc_linear_body(x_ref, w_ref, c0_ref, c1_ref, hp0_ref, hp1_ref, dis_ref):
    deg = c0_ref[...] + c1_ref[...] + 1.0
    dis = lax.rsqrt(deg)
    dis_ref[...] = dis
    h = jnp.dot(x_ref[...], w_ref[...], preferred_element_type=jnp.float32)
    hp = h * dis
    hp0_ref[...] = hp[:, :HD]
    hp1_ref[...] = hp[:, HD:]


_tc_linear = pl.pallas_call(
    _tc_linear_body,
    grid=(GRID,),
    in_specs=[
        pl.BlockSpec((BR, D), lambda i: (i, 0)),
        pl.BlockSpec((D, D), lambda i: (0, 0)),
        pl.BlockSpec((BR, 1), lambda i: (i, 0)),
        pl.BlockSpec((BR, 1), lambda i: (i, 0)),
    ],
    out_specs=[
        pl.BlockSpec((BR, HD), lambda i: (i, 0)),
        pl.BlockSpec((BR, HD), lambda i: (i, 0)),
        pl.BlockSpec((BR, 1), lambda i: (i, 0)),
    ],
    out_shape=[
        jax.ShapeDtypeStruct((NP, HD), jnp.float32),
        jax.ShapeDtypeStruct((NP, HD), jnp.float32),
        jax.ShapeDtypeStruct((NP, 1), jnp.float32),
    ],
)


def _tc_mlp_body(hp0_ref, hp1_ref, agg_ref, dis_ref, w1_ref, b1_ref,
                 w2_ref, b2_ref, out_ref):
    dis = dis_ref[...]
    t0 = (hp0_ref[...] + agg_ref[0]) * dis
    t1 = (hp1_ref[...] + agg_ref[1]) * dis
    t = jnp.concatenate([t0, t1], axis=1)
    z = jnp.dot(t, w1_ref[...], preferred_element_type=jnp.float32) + b1_ref[...]
    z = jnp.maximum(z, 0.0)
    out_ref[...] = (jnp.dot(z, w2_ref[...], preferred_element_type=jnp.float32)
                    + b2_ref[...])


_tc_mlp = pl.pallas_call(
    _tc_mlp_body,
    grid=(GRID,),
    in_specs=[
        pl.BlockSpec((BR, HD), lambda i: (i, 0)),
        pl.BlockSpec((BR, HD), lambda i: (i, 0)),
        pl.BlockSpec((2, BR, HD), lambda i: (0, i, 0)),
        pl.BlockSpec((BR, 1), lambda i: (i, 0)),
        pl.BlockSpec((D, D), lambda i: (0, 0)),
        pl.BlockSpec((1, D), lambda i: (0, 0)),
        pl.BlockSpec((D, D), lambda i: (0, 0)),
        pl.BlockSpec((1, D), lambda i: (0, 0)),
    ],
    out_specs=pl.BlockSpec((BR, D), lambda i: (i, 0)),
    out_shape=jax.ShapeDtypeStruct((NP, D), jnp.float32),
)


def kernel(x, edge_index, W_gcn, W1, b1, W2, b2):
    row = edge_index[0]
    col = edge_index[1]
    row_a = jnp.concatenate(
        [row, jnp.full((E_A - E,), N, jnp.int32)]).reshape(32, 40, 128)
    row_b = jnp.concatenate(
        [row, jnp.full((E_B - E,), N, jnp.int32)]).reshape(16, 79, 128)
    col_b = jnp.concatenate(
        [col, jnp.zeros((E_B - E,), jnp.int32)]).reshape(16, 79, 128)
    x_pad = jnp.pad(x, ((0, NP - N), (0, 0)))

    counts2 = _sc_counts(row_a)
    c0 = counts2[0].reshape(NP, 1)
    c1 = counts2[1].reshape(NP, 1)
    hp0, hp1, dis = _tc_linear(x_pad, W_gcn, c0, c1)
    agg = _sc_spmm(hp0, hp1, row_b, col_b)
    out = _tc_mlp(hp0, hp1, agg, dis, W1, b1.reshape(1, D), W2,
                  b2.reshape(1, D))
    return out[:N]


# speedup vs baseline: 12.9597x; 12.9597x over previous
import functools

import jax
import jax.numpy as jnp
from jax import lax
from jax.experimental import pallas as pl
from jax.experimental.pallas import tpu as pltpu
from jax.experimental.pallas import tpu_sc as plsc

N = 10000
NP = 10240
D = 256
HD = 128
E = 160000
E_A = 163840
E_B = 161792
BR = 1024
GRID = NP // BR

@functools.cache
def _get_sc_counts():
    mesh = plsc.VectorSubcoreMesh(core_axis_name="c", subcore_axis_name="s")
    return pl.kernel(
        _sc_counts_body,
        out_type=jax.ShapeDtypeStruct((2, NP), jnp.float32),
        mesh=mesh,
        scratch_types=[
            pltpu.VMEM((40, 128), jnp.int32),
            pltpu.VMEM((128,), jnp.float32),
            pltpu.VMEM((640,), jnp.float32),
            pltpu.VMEM_SHARED((NP,), jnp.float32),
        ],
    )


def _sc_counts_body(row_hbm, out_hbm, idx_v, ones_v, z_v, counts_sh):
    c = lax.axis_index("c")
    s = lax.axis_index("s")
    wid = s * 2 + c

    @pl.loop(0, 640, step=16)
    def _(k):
        z_v[pl.ds(k, 16)] = jnp.zeros((16,), jnp.float32)

    @pl.loop(0, 128, step=16)
    def _(k):
        ones_v[pl.ds(k, 16)] = jnp.ones((16,), jnp.float32)

    pltpu.sync_copy(z_v, counts_sh.at[pl.ds(s * 640, 640)])
    plsc.subcore_barrier()

    pltpu.sync_copy(row_hbm.at[wid], idx_v)

    @pl.loop(0, 40)
    def _(j):
        pltpu.sync_copy(ones_v, counts_sh.at[idx_v.at[j]], add=True)

    plsc.subcore_barrier()
    pltpu.sync_copy(counts_sh.at[pl.ds(s * 640, 640)],
                    out_hbm.at[c].at[pl.ds(s * 640, 640)])


@functools.cache
def _get_sc_spmm():
    mesh = plsc.VectorSubcoreMesh(core_axis_name="c", subcore_axis_name="s")
    return pl.kernel(
        _sc_spmm_body,
        out_type=jax.ShapeDtypeStruct((2, NP, HD), jnp.float32),
        mesh=mesh,
        scratch_types=[
            pltpu.VMEM((79, 128), jnp.int32),
            pltpu.VMEM((79, 128), jnp.int32),
            pltpu.VMEM((128, HD), jnp.float32),
            pltpu.VMEM((64, HD), jnp.float32),
            pltpu.VMEM_SHARED((NP, HD), jnp.float32),
        ],
    )


def _sc_spmm_body(hp0_hbm, hp1_hbm, row_hbm, col_hbm, out_hbm,
                  ridx, cidx, gbuf, zbuf, accum):
    c = lax.axis_index("c")
    s = lax.axis_index("s")

    @pl.loop(0, 64)
    def _(r):
        @pl.loop(0, HD, step=16)
        def _(k):
            zbuf[r, pl.ds(k, 16)] = jnp.zeros((16,), jnp.float32)

    @pl.loop(0, 10)
    def _(t):
        pltpu.sync_copy(zbuf, accum.at[pl.ds(s * 640 + t * 64, 64)])

    plsc.subcore_barrier()

    pltpu.sync_copy(row_hbm.at[s], ridx)
    pltpu.sync_copy(col_hbm.at[s], cidx)

    def run_half(hp_hbm):
        @pl.loop(0, 79)
        def _(j):
            pltpu.sync_copy(hp_hbm.at[cidx.at[j]], gbuf)
            pltpu.sync_copy(gbuf, accum.at[ridx.at[j]], add=True)

    @pl.when(c == 0)
    def _():
        run_half(hp0_hbm)

    @pl.when(c == 1)
    def _():
        run_half(hp1_hbm)

    plsc.subcore_barrier()
    pltpu.sync_copy(accum.at[pl.ds(s * 640, 640)],
                    out_hbm.at[c].at[pl.ds(s * 640, 640)])


def _tc_linear_body(x_ref, w_ref, c0_ref, c1_ref, hp0_ref, hp1_ref, dis_ref):
    deg = c0_ref[...] + c1_ref[...] + 1.0
    dis = lax.rsqrt(deg)
    dis_ref[...] = dis
    h = jnp.dot(x_ref[...], w_ref[...], preferred_element_type=jnp.float32)
    hp = h * dis
    hp0_ref[...] = hp[:, :HD]
    hp1_ref[...] = hp[:, HD:]


_tc_linear = pl.pallas_call(
    _tc_linear_body,
    grid=(GRID,),
    in_specs=[
        pl.BlockSpec((BR, D), lambda i: (i, 0)),
        pl.BlockSpec((D, D), lambda i: (0, 0)),
        pl.BlockSpec((BR, 1), lambda i: (i, 0)),
        pl.BlockSpec((BR, 1), lambda i: (i, 0)),
    ],
    out_specs=[
        pl.BlockSpec((BR, HD), lambda i: (i, 0)),
        pl.BlockSpec((BR, HD), lambda i: (i, 0)),
        pl.BlockSpec((BR, 1), lambda i: (i, 0)),
    ],
    out_shape=[
        jax.ShapeDtypeStruct((NP, HD), jnp.float32),
        jax.ShapeDtypeStruct((NP, HD), jnp.float32),
        jax.ShapeDtypeStruct((NP, 1), jnp.float32),
    ],
)


def _tc_mlp_body(hp0_ref, hp1_ref, agg_ref, dis_ref, w1_ref, b1_ref,
                 w2_ref, b2_ref, out_ref):
    dis = dis_ref[...]
    t0 = (hp0_ref[...] + agg_ref[0]) * dis
    t1 = (hp1_ref[...] + agg_ref[1]) * dis
    t = jnp.concatenate([t0, t1], axis=1)
    z = jnp.dot(t, w1_ref[...], preferred_element_type=jnp.float32) + b1_ref[...]
    z = jnp.maximum(z, 0.0)
    out_ref[...] = (jnp.dot(z, w2_ref[...], preferred_element_type=jnp.float32)
                    + b2_ref[...])


_tc_mlp = pl.pallas_call(
    _tc_mlp_body,
    grid=(GRID,),
    in_specs=[
        pl.BlockSpec((BR, HD), lambda i: (i, 0)),
        pl.BlockSpec((BR, HD), lambda i: (i, 0)),
        pl.BlockSpec((2, BR, HD), lambda i: (0, i, 0)),
        pl.BlockSpec((BR, 1), lambda i: (i, 0)),
        pl.BlockSpec((D, D), lambda i: (0, 0)),
        pl.BlockSpec((1, D), lambda i: (0, 0)),
        pl.BlockSpec((D, D), lambda i: (0, 0)),
        pl.BlockSpec((1, D), lambda i: (0, 0)),
    ],
    out_specs=pl.BlockSpec((BR, D), lambda i: (i, 0)),
    out_shape=jax.ShapeDtypeStruct((NP, D), jnp.float32),
)


def kernel(x, edge_index, W_gcn, W1, b1, W2, b2):
    row = edge_index[0]
    col = edge_index[1]
    row_a = jnp.concatenate(
        [row, jnp.full((E_A - E,), N, jnp.int32)]).reshape(32, 40, 128)
    row_b = jnp.concatenate(
        [row, jnp.full((E_B - E,), N, jnp.int32)]).reshape(16, 79, 128)
    col_b = jnp.concatenate(
        [col, jnp.zeros((E_B - E,), jnp.int32)]).reshape(16, 79, 128)
    x_pad = jnp.pad(x, ((0, NP - N), (0, 0)))

    counts2 = _get_sc_counts()(row_a)
    c0 = counts2[0].reshape(NP, 1)
    c1 = counts2[1].reshape(NP, 1)
    hp0, hp1, dis = _tc_linear(x_pad, W_gcn, c0, c1)
    agg = _get_sc_spmm()(hp0, hp1, row_b, col_b)
    out = _tc_mlp(hp0, hp1, agg, dis, W1, b1.reshape(1, D), W2,
                  b2.reshape(1, D))
    return out[:N]
